# baseline (speedup 1.0000x reference)
import functools

import jax
import jax.numpy as jnp
from jax import lax
from jax.experimental import pallas as pl
from jax.experimental.pallas import tpu as pltpu
from jax.experimental.pallas import tpu_sc as plsc

_NC = 2
_NS = 16
_NW = _NC * _NS
_L = 16


@functools.lru_cache(maxsize=None)
def _make_gather(VQ, B, C):
    assert B % (_NW * C) == 0
    rows_per_worker = B // _NW
    num_chunks = rows_per_worker // C
    mesh = plsc.VectorSubcoreMesh(core_axis_name="c", subcore_axis_name="s")

    @functools.partial(
        pl.kernel,
        mesh=mesh,
        out_type=jax.ShapeDtypeStruct((B, 32), jnp.float32),
        scratch_types=[
            pltpu.VMEM((C,), jnp.int32),
            pltpu.VMEM((C,), jnp.int32),
            pltpu.VMEM((C,), jnp.int32),
            pltpu.VMEM((C, 128), jnp.float32),
            pltpu.VMEM((C, 32), jnp.float32),
            pltpu.SemaphoreType.DMA,
        ],
        compiler_params=pltpu.CompilerParams(needs_layout_passes=False),
    )
    def k(t128_hbm, idx_hbm, out_hbm, idx_v, q_v, sb_v, rows_v, out_v, sem):
        wid = lax.axis_index("s") * _NC + lax.axis_index("c")
        base = wid * rows_per_worker
        iota = lax.iota(jnp.int32, _L)

        def chunk_body(j, carry):
            off = base + j * C
            pltpu.sync_copy(idx_hbm.at[pl.ds(off, C)], idx_v)

            def prep(g, c2):
                v = idx_v[pl.ds(g * _L, _L)]
                q_v[pl.ds(g * _L, _L)] = lax.shift_right_logical(v, 2)
                sb_v[pl.ds(g * _L, _L)] = lax.shift_left(
                    lax.bitwise_and(v, 3), 5)
                return c2

            lax.fori_loop(0, C // _L, prep, 0)

            pltpu.async_copy(t128_hbm.at[q_v], rows_v, sem).wait()

            def extract(g, c2):
                rvec = iota + g * _L
                sb16 = sb_v[pl.ds(g * _L, _L)]
                for c in range(32):
                    vals = plsc.load_gather(rows_v, [rvec, sb16 + c])
                    plsc.store_scatter(
                        out_v, [rvec, jnp.full((_L,), c, jnp.int32)], vals)
                return c2

            lax.fori_loop(0, C // _L, extract, 0)

            pltpu.sync_copy(out_v, out_hbm.at[pl.ds(off, C)])
            return carry

        lax.fori_loop(0, num_chunks, chunk_body, 0)

    return k


def kernel(weight, indices):
    V, D = weight.shape
    B = indices.size
    t128 = weight.reshape(V * D // 128, 128)
    idx = indices.reshape(-1).astype(jnp.int32)
    out = _make_gather(t128.shape[0], B, 256)(t128, idx)
    return out.reshape(indices.shape + (D,))


# speedup vs baseline: 1.6575x; 1.6575x over previous
import functools

import jax
import jax.numpy as jnp
from jax import lax
from jax.experimental import pallas as pl
from jax.experimental.pallas import tpu as pltpu
from jax.experimental.pallas import tpu_sc as plsc

_NC = 2
_NS = 16
_NW = _NC * _NS


@functools.lru_cache(maxsize=None)
def _make_gather(V, D, B, C):
    assert B % (_NW * C) == 0
    rows_per_worker = B // _NW
    num_chunks = rows_per_worker // C
    mesh = plsc.VectorSubcoreMesh(core_axis_name="c", subcore_axis_name="s")

    @functools.partial(
        pl.kernel,
        mesh=mesh,
        out_type=jax.ShapeDtypeStruct((B, D), jnp.float32),
        scratch_types=[
            pltpu.VMEM((C,), jnp.int32),
            pltpu.VMEM((C,), jnp.int32),
            pltpu.VMEM((C, D), jnp.float32),
            pltpu.VMEM((C, D), jnp.float32),
            pltpu.SemaphoreType.DMA,
            pltpu.SemaphoreType.DMA,
            pltpu.SemaphoreType.DMA,
            pltpu.SemaphoreType.DMA,
            pltpu.SemaphoreType.DMA,
            pltpu.SemaphoreType.DMA,
        ],
        compiler_params=pltpu.CompilerParams(use_tc_tiling_on_sc=False),
    )
    def k(t_hbm, idx_hbm, out_hbm,
          i0, i1, r0, r1, si0, si1, sg0, sg1, so0, so1):
        idx_v = (i0, i1)
        rows_v = (r0, r1)
        sem_i = (si0, si1)
        sem_o = (so0, so1)
        sem_g = (sg0, sg1)
        wid = lax.axis_index("s") * _NC + lax.axis_index("c")
        base = wid * rows_per_worker

        idx_d = {}
        out_d = {}
        for j in range(min(2, num_chunks)):
            idx_d[j] = pltpu.async_copy(
                idx_hbm.at[pl.ds(base + j * C, C)], idx_v[j % 2],
                sem_i[j % 2])
        for j in range(num_chunks):
            b = j % 2
            idx_d[j].wait()
            if j >= 2:
                out_d[j - 2].wait()
            pltpu.async_copy(t_hbm.at[idx_v[b]], rows_v[b], sem_g[b]).wait()
            out_d[j] = pltpu.async_copy(
                rows_v[b], out_hbm.at[pl.ds(base + j * C, C)], sem_o[b])
            if j + 2 < num_chunks:
                idx_d[j + 2] = pltpu.async_copy(
                    idx_hbm.at[pl.ds(base + (j + 2) * C, C)], idx_v[b],
                    sem_i[b])
        for j in range(max(0, num_chunks - 2), num_chunks):
            out_d[j].wait()

    return k


def kernel(weight, indices):
    V, D = weight.shape
    B = indices.size
    idx = indices.reshape(-1).astype(jnp.int32)
    out = _make_gather(V, D, B, 1600)(weight, idx)
    return out.reshape(indices.shape + (D,))
